# baseline (speedup 1.0000x reference)
import functools

import jax
import jax.numpy as jnp
from jax import lax
from jax.experimental import pallas as pl
from jax.experimental.pallas import tpu as pltpu
from jax.experimental.pallas import tpu_sc as plsc

_NUM_BINS = 1000000
_EMB = 64
_NC = 2
_NS = 16
_NW = _NC * _NS
_CHUNK = 128


def _sc_user_gather(table, idx):
    B = idx.shape[0]
    D = table.shape[1]
    b_per_w = B // _NW
    group = 16
    n_groups = b_per_w // group
    mesh = plsc.VectorSubcoreMesh(core_axis_name="c", subcore_axis_name="s")

    @functools.partial(
        pl.kernel,
        mesh=mesh,
        out_type=jax.ShapeDtypeStruct((B, D), table.dtype),
        scratch_types=[
            pltpu.VMEM((b_per_w,), jnp.int32),
            pltpu.VMEM((b_per_w, D), table.dtype),
            pltpu.SemaphoreType.DMA,
        ],
        compiler_params=pltpu.CompilerParams(use_tc_tiling_on_sc=True),
    )
    def gather_kernel(idx_hbm, table_hbm, out_hbm, idx_v, rows_v, sem):
        wid = lax.axis_index("s") * _NC + lax.axis_index("c")
        base = wid * b_per_w
        pltpu.sync_copy(idx_hbm.at[pl.ds(base, b_per_w)], idx_v)

        def body(g, _):
            vec = idx_v[pl.ds(g * group, group)]
            copies = []
            for t in range(group):
                i = g * group + t
                r = vec[t]
                copies.append(
                    pltpu.async_copy(
                        table_hbm.at[pl.ds(r, 1)],
                        rows_v.at[pl.ds(i, 1)],
                        sem,
                    )
                )
            for c in copies:
                c.wait()
            return _

        lax.fori_loop(0, n_groups, body, None)
        pltpu.sync_copy(rows_v, out_hbm.at[pl.ds(base, b_per_w)])

    return gather_kernel(idx, table)


def _tc_body(uref, pref, iref, at, gt, wt, ht, mt, st, it, ct, ckr, fwr, fbr,
             w1r, b1r, w2r, b2r, w3r, b3r, w4r, b4r, w5r, b5r, outr):
    T = uref.shape[0]
    f32 = jnp.float32
    tabs = [at, gt, wt, ht, mt, st, it, ct]
    mods = [None, 4, None, None, None, None, 200, None]
    u2 = uref[...]
    par = (pref[...] > 0).astype(f32)
    u = u2[:, 64:128] * par + u2[:, 0:64] * (1.0 - par)
    embs = [u]
    for f in range(8):
        idx = iref[:, f:f + 1]
        if mods[f] is not None:
            idx = lax.rem(idx, mods[f])
        rows = tabs[f].shape[0]
        oh = (idx == lax.broadcasted_iota(jnp.int32, (T, rows), 1)).astype(f32)
        embs.append(jnp.dot(oh, tabs[f][...], preferred_element_type=f32))
    e = jnp.concatenate(embs, axis=1)

    bf16 = jnp.bfloat16

    def bdot(x, y):
        return jnp.dot(x.astype(bf16), y.astype(bf16), preferred_element_type=f32)

    K = ckr[...]
    a = bdot(e, K)
    asum = jnp.sum(a * a, axis=1, keepdims=True)
    srow = jnp.sum(K * K, axis=1, keepdims=True)
    bsum = jnp.dot(e * e, srow, preferred_element_type=f32)
    cross = (0.5 / e.shape[1]) * (asum - bsum)
    linear = jnp.dot(e, fwr[...], preferred_element_type=f32) + fbr[0, 0]

    h = jnp.maximum(bdot(e, w1r[...]) + b1r[...], 0.0)
    h = jnp.maximum(bdot(h, w2r[...]) + b2r[...], 0.0)
    h = jnp.maximum(bdot(h, w3r[...]) + b3r[...], 0.0)
    h = jnp.maximum(bdot(h, w4r[...]) + b4r[...], 0.0)
    dnn = jnp.dot(h, w5r[...], preferred_element_type=f32) + b5r[0, 0]

    logit = linear + cross + dnn
    outr[...] = jax.nn.sigmoid(logit)


def kernel(user_id, age, gender, weekday, hour, minute, second, item_id,
           item_catalog, user_table, age_table, gender_table, weekday_table,
           hour_table, minute_table, second_table, item_table, catalog_table,
           fm_w, fm_b, cross_k, w1, b1, w2, b2, w3, b3, w4, b4, w5, b5):
    B = user_id.shape[0]
    u_idx = jnp.mod(user_id, _NUM_BINS).astype(jnp.int32)
    table2 = user_table.reshape(_NUM_BINS // 2, 2 * _EMB)
    user_pairs = _sc_user_gather(table2, u_idx // 2)
    u_par = jnp.remainder(u_idx, 2).reshape(-1, 1)

    sidx = jnp.stack(
        [age, gender, weekday, hour, minute, second, item_id, item_catalog],
        axis=1,
    ).astype(jnp.int32)

    T = 512
    grid = (B // T,)
    small_tables = [age_table, gender_table, weekday_table, hour_table,
                    minute_table, second_table, item_table, catalog_table]

    def rep(shape):
        nd = len(shape)
        return pl.BlockSpec(shape, lambda i, _n=nd: (0,) * _n)

    in_specs = [
        pl.BlockSpec((T, 2 * _EMB), lambda i: (i, 0)),
        pl.BlockSpec((T, 1), lambda i: (i, 0)),
        pl.BlockSpec((T, 8), lambda i: (i, 0)),
        *[rep(t.shape) for t in small_tables],
        rep(cross_k.shape),
        rep(fm_w.shape), rep((1, 1)),
        rep(w1.shape), rep((1, w1.shape[1])),
        rep(w2.shape), rep((1, w2.shape[1])),
        rep(w3.shape), rep((1, w3.shape[1])),
        rep(w4.shape), rep((1, w4.shape[1])),
        rep(w5.shape), rep((1, 1)),
    ]

    out = pl.pallas_call(
        _tc_body,
        grid=grid,
        in_specs=in_specs,
        out_specs=pl.BlockSpec((T, 1), lambda i: (i, 0)),
        out_shape=jax.ShapeDtypeStruct((B, 1), jnp.float32),
    )(user_pairs, u_par, sidx, *small_tables, cross_k, fm_w, fm_b.reshape(1, 1),
      w1, b1.reshape(1, -1), w2, b2.reshape(1, -1), w3, b3.reshape(1, -1),
      w4, b4.reshape(1, -1), w5, b5.reshape(1, -1))
    return out


# speedup vs baseline: 1.6176x; 1.6176x over previous
import functools

import jax
import jax.numpy as jnp
from jax import lax
from jax.experimental import pallas as pl
from jax.experimental.pallas import tpu as pltpu
from jax.experimental.pallas import tpu_sc as plsc

_NUM_BINS = 1000000
_EMB = 64
_NC = 2
_NS = 16
_NW = _NC * _NS
_CHUNK = 128


def _sc_user_gather(table, idx):
    B = idx.shape[0]
    D = table.shape[1]
    b_per_w = B // _NW
    group = 16
    n_groups = b_per_w // group
    mesh = plsc.VectorSubcoreMesh(core_axis_name="c", subcore_axis_name="s")

    @functools.partial(
        pl.kernel,
        mesh=mesh,
        out_type=jax.ShapeDtypeStruct((B, D), table.dtype),
        scratch_types=[
            pltpu.VMEM((b_per_w,), jnp.int32),
            pltpu.VMEM((b_per_w, D), table.dtype),
            pltpu.SemaphoreType.DMA,
        ],
        compiler_params=pltpu.CompilerParams(use_tc_tiling_on_sc=True),
    )
    def gather_kernel(idx_hbm, table_hbm, out_hbm, idx_v, rows_v, sem):
        wid = lax.axis_index("s") * _NC + lax.axis_index("c")
        base = wid * b_per_w
        pltpu.sync_copy(idx_hbm.at[pl.ds(base, b_per_w)], idx_v)

        def body(g, _):
            vec = idx_v[pl.ds(g * group, group)]
            copies = []
            for t in range(group):
                i = g * group + t
                r = vec[t]
                copies.append(
                    pltpu.async_copy(
                        table_hbm.at[pl.ds(r, 1)],
                        rows_v.at[pl.ds(i, 1)],
                        sem,
                    )
                )
            for c in copies:
                c.wait()
            return _

        lax.fori_loop(0, n_groups, body, None)
        pltpu.sync_copy(rows_v, out_hbm.at[pl.ds(base, b_per_w)])

    return gather_kernel(idx, table)


def _tc_body(uref, iref, at, gt, wt, ht, mt, st, it, ct, ckr, fwr, fbr,
             w1r, b1r, w2r, b2r, w3r, b3r, w4r, b4r, w5r, b5r, outr):
    T = uref.shape[0]
    f32 = jnp.float32
    tabs = [at, gt, wt, ht, mt, st, it, ct]
    mods = [None, 4, None, None, None, None, 200, None]
    embs = [uref[...]]
    for f in range(8):
        idx = iref[:, f:f + 1]
        if mods[f] is not None:
            idx = lax.rem(idx, mods[f])
        rows = tabs[f].shape[0]
        oh = (idx == lax.broadcasted_iota(jnp.int32, (T, rows), 1)).astype(f32)
        embs.append(jnp.dot(oh, tabs[f][...], preferred_element_type=f32))
    e = jnp.concatenate(embs, axis=1)

    bf16 = jnp.bfloat16

    def bdot(x, y):
        return jnp.dot(x.astype(bf16), y.astype(bf16), preferred_element_type=f32)

    K = ckr[...]
    a = bdot(e, K)
    asum = jnp.sum(a * a, axis=1, keepdims=True)
    srow = jnp.sum(K * K, axis=1, keepdims=True)
    bsum = jnp.dot(e * e, srow, preferred_element_type=f32)
    cross = (0.5 / e.shape[1]) * (asum - bsum)
    linear = jnp.dot(e, fwr[...], preferred_element_type=f32) + fbr[0, 0]

    h = jnp.maximum(bdot(e, w1r[...]) + b1r[...], 0.0)
    h = jnp.maximum(bdot(h, w2r[...]) + b2r[...], 0.0)
    h = jnp.maximum(bdot(h, w3r[...]) + b3r[...], 0.0)
    h = jnp.maximum(bdot(h, w4r[...]) + b4r[...], 0.0)
    dnn = jnp.dot(h, w5r[...], preferred_element_type=f32) + b5r[0, 0]

    logit = linear + cross + dnn
    outr[...] = jax.nn.sigmoid(logit)


def kernel(user_id, age, gender, weekday, hour, minute, second, item_id,
           item_catalog, user_table, age_table, gender_table, weekday_table,
           hour_table, minute_table, second_table, item_table, catalog_table,
           fm_w, fm_b, cross_k, w1, b1, w2, b2, w3, b3, w4, b4, w5, b5):
    B = user_id.shape[0]
    u_idx = jnp.mod(user_id, _NUM_BINS).astype(jnp.int32)
    user_emb = _sc_user_gather(user_table, u_idx)

    sidx = jnp.stack(
        [age, gender, weekday, hour, minute, second, item_id, item_catalog],
        axis=1,
    ).astype(jnp.int32)

    T = 1024
    grid = (B // T,)
    small_tables = [age_table, gender_table, weekday_table, hour_table,
                    minute_table, second_table, item_table, catalog_table]

    def rep(shape):
        nd = len(shape)
        return pl.BlockSpec(shape, lambda i, _n=nd: (0,) * _n)

    in_specs = [
        pl.BlockSpec((T, _EMB), lambda i: (i, 0)),
        pl.BlockSpec((T, 8), lambda i: (i, 0)),
        *[rep(t.shape) for t in small_tables],
        rep(cross_k.shape),
        rep(fm_w.shape), rep((1, 1)),
        rep(w1.shape), rep((1, w1.shape[1])),
        rep(w2.shape), rep((1, w2.shape[1])),
        rep(w3.shape), rep((1, w3.shape[1])),
        rep(w4.shape), rep((1, w4.shape[1])),
        rep(w5.shape), rep((1, 1)),
    ]

    out = pl.pallas_call(
        _tc_body,
        grid=grid,
        in_specs=in_specs,
        out_specs=pl.BlockSpec((T, 1), lambda i: (i, 0)),
        out_shape=jax.ShapeDtypeStruct((B, 1), jnp.float32),
    )(user_emb, sidx, *small_tables, cross_k, fm_w, fm_b.reshape(1, 1),
      w1, b1.reshape(1, -1), w2, b2.reshape(1, -1), w3, b3.reshape(1, -1),
      w4, b4.reshape(1, -1), w5, b5.reshape(1, -1))
    return out


# speedup vs baseline: 1.6487x; 1.0192x over previous
import functools

import jax
import jax.numpy as jnp
from jax import lax
from jax.experimental import pallas as pl
from jax.experimental.pallas import tpu as pltpu
from jax.experimental.pallas import tpu_sc as plsc

_NUM_BINS = 1000000
_EMB = 64
_NC = 2
_NS = 16
_NW = _NC * _NS
_CHUNK = 128


def _sc_user_gather(table, idx):
    B = idx.shape[0]
    D = table.shape[1]
    b_per_w = B // _NW
    group = 16
    n_groups = b_per_w // group
    mesh = plsc.VectorSubcoreMesh(core_axis_name="c", subcore_axis_name="s")

    @functools.partial(
        pl.kernel,
        mesh=mesh,
        out_type=jax.ShapeDtypeStruct((B, D), table.dtype),
        scratch_types=[
            pltpu.VMEM((b_per_w,), jnp.int32),
            pltpu.VMEM((b_per_w, D), table.dtype),
            pltpu.SemaphoreType.DMA,
        ],
        compiler_params=pltpu.CompilerParams(use_tc_tiling_on_sc=True),
    )
    def gather_kernel(idx_hbm, table_hbm, out_hbm, idx_v, rows_v, sem):
        wid = lax.axis_index("s") * _NC + lax.axis_index("c")
        base = wid * b_per_w
        pltpu.sync_copy(idx_hbm.at[pl.ds(base, b_per_w)], idx_v)

        def body(g, _):
            copies = []
            for half in range(2):
                vec = idx_v[pl.ds((2 * g + half) * group, group)]
                for t in range(group):
                    i = (2 * g + half) * group + t
                    r = vec[t]
                    copies.append(
                        pltpu.async_copy(
                            table_hbm.at[pl.ds(r, 1)],
                            rows_v.at[pl.ds(i, 1)],
                            sem,
                        )
                    )
            for c in copies:
                c.wait()
            return _

        lax.fori_loop(0, n_groups // 2, body, None)
        pltpu.sync_copy(rows_v, out_hbm.at[pl.ds(base, b_per_w)])

    return gather_kernel(idx, table)


def _tc_body(uref, iref, at, gt, wt, ht, mt, st, it, ct, ckr, fwr, fbr,
             w1r, b1r, w2r, b2r, w3r, b3r, w4r, b4r, w5r, b5r, outr):
    T = uref.shape[0]
    f32 = jnp.float32
    tabs = [at, gt, wt, ht, mt, st, it, ct]
    mods = [None, 4, None, None, None, None, 200, None]
    embs = [uref[...]]
    for f in range(8):
        idx = iref[:, f:f + 1]
        if mods[f] is not None:
            idx = lax.rem(idx, mods[f])
        rows = tabs[f].shape[0]
        oh = (idx == lax.broadcasted_iota(jnp.int32, (T, rows), 1)).astype(f32)
        embs.append(jnp.dot(oh, tabs[f][...], preferred_element_type=f32))
    e = jnp.concatenate(embs, axis=1)

    bf16 = jnp.bfloat16

    def bdot(x, y):
        return jnp.dot(x.astype(bf16), y.astype(bf16), preferred_element_type=f32)

    K = ckr[...]
    a = bdot(e, K)
    asum = jnp.sum(a * a, axis=1, keepdims=True)
    srow = jnp.sum(K * K, axis=1, keepdims=True)
    bsum = jnp.dot(e * e, srow, preferred_element_type=f32)
    cross = (0.5 / e.shape[1]) * (asum - bsum)
    linear = jnp.dot(e, fwr[...], preferred_element_type=f32) + fbr[0, 0]

    h = jnp.maximum(bdot(e, w1r[...]) + b1r[...], 0.0)
    h = jnp.maximum(bdot(h, w2r[...]) + b2r[...], 0.0)
    h = jnp.maximum(bdot(h, w3r[...]) + b3r[...], 0.0)
    h = jnp.maximum(bdot(h, w4r[...]) + b4r[...], 0.0)
    dnn = jnp.dot(h, w5r[...], preferred_element_type=f32) + b5r[0, 0]

    logit = linear + cross + dnn
    outr[...] = jax.nn.sigmoid(logit)


def kernel(user_id, age, gender, weekday, hour, minute, second, item_id,
           item_catalog, user_table, age_table, gender_table, weekday_table,
           hour_table, minute_table, second_table, item_table, catalog_table,
           fm_w, fm_b, cross_k, w1, b1, w2, b2, w3, b3, w4, b4, w5, b5):
    B = user_id.shape[0]
    u_idx = jnp.mod(user_id, _NUM_BINS).astype(jnp.int32)
    user_emb = _sc_user_gather(user_table, u_idx)

    sidx = jnp.stack(
        [age, gender, weekday, hour, minute, second, item_id, item_catalog],
        axis=1,
    ).astype(jnp.int32)

    T = 1024
    grid = (B // T,)
    small_tables = [age_table, gender_table, weekday_table, hour_table,
                    minute_table, second_table, item_table, catalog_table]

    def rep(shape):
        nd = len(shape)
        return pl.BlockSpec(shape, lambda i, _n=nd: (0,) * _n)

    in_specs = [
        pl.BlockSpec((T, _EMB), lambda i: (i, 0)),
        pl.BlockSpec((T, 8), lambda i: (i, 0)),
        *[rep(t.shape) for t in small_tables],
        rep(cross_k.shape),
        rep(fm_w.shape), rep((1, 1)),
        rep(w1.shape), rep((1, w1.shape[1])),
        rep(w2.shape), rep((1, w2.shape[1])),
        rep(w3.shape), rep((1, w3.shape[1])),
        rep(w4.shape), rep((1, w4.shape[1])),
        rep(w5.shape), rep((1, 1)),
    ]

    out = pl.pallas_call(
        _tc_body,
        grid=grid,
        in_specs=in_specs,
        out_specs=pl.BlockSpec((T, 1), lambda i: (i, 0)),
        out_shape=jax.ShapeDtypeStruct((B, 1), jnp.float32),
    )(user_emb, sidx, *small_tables, cross_k, fm_w, fm_b.reshape(1, 1),
      w1, b1.reshape(1, -1), w2, b2.reshape(1, -1), w3, b3.reshape(1, -1),
      w4, b4.reshape(1, -1), w5, b5.reshape(1, -1))
    return out
